# initial kernel scaffold (unmeasured)
import jax
import jax.numpy as jnp
from jax import lax
from jax.experimental import pallas as pl
from jax.experimental.pallas import tpu as pltpu

N_Y = 4
B = 2
S_PER = 512
H = 8
D = 64
SCALE = D ** -0.5


def kernel(Q, K, V):
    def body(q_ref, k_ref, v_ref, out_ref,
             kfull, vfull, ksend, krecv, vsend, vrecv):
        my_x = lax.axis_index("x")
        my_y = lax.axis_index("y")
        my_z = lax.axis_index("z")
        right = (my_y + 1) % N_Y
        left = (my_y + N_Y - 1) % N_Y

        barrier = pltpu.get_barrier_semaphore()
        for nbr in (left, right):
            pl.semaphore_signal(
                barrier, inc=1,
                device_id=(my_x, nbr, my_z),
                device_id_type=pl.DeviceIdType.MESH,
            )
        pl.semaphore_wait(barrier, 2)

        kfull[pl.ds(my_y, 1)] = k_ref[...].astype(jnp.bfloat16)[None]
        vfull[pl.ds(my_y, 1)] = v_ref[...].astype(jnp.bfloat16)[None]

        for h in range(N_Y - 1):
            k_src = (my_y + N_Y - h) % N_Y
            k_rcv = (my_y + N_Y - h - 1) % N_Y
            v_src = (my_y + h) % N_Y
            v_rcv = (my_y + h + 1) % N_Y

            k_send_rdma = pltpu.make_async_remote_copy(
                src_ref=kfull.at[k_src],
                dst_ref=kfull.at[k_src],
                send_sem=ksend.at[h],
                recv_sem=krecv.at[h],
                device_id=(my_x, right, my_z),
                device_id_type=pl.DeviceIdType.MESH,
            )
            v_send_rdma = pltpu.make_async_remote_copy(
                src_ref=vfull.at[v_src],
                dst_ref=vfull.at[v_src],
                send_sem=vsend.at[h],
                recv_sem=vrecv.at[h],
                device_id=(my_x, left, my_z),
                device_id_type=pl.DeviceIdType.MESH,
            )
            k_send_rdma.start()
            v_send_rdma.start()

            k_recv_rdma = pltpu.make_async_remote_copy(
                src_ref=kfull.at[k_rcv],
                dst_ref=kfull.at[k_rcv],
                send_sem=ksend.at[h],
                recv_sem=krecv.at[h],
                device_id=(my_x, left, my_z),
                device_id_type=pl.DeviceIdType.MESH,
            )
            v_recv_rdma = pltpu.make_async_remote_copy(
                src_ref=vfull.at[v_rcv],
                dst_ref=vfull.at[v_rcv],
                send_sem=vsend.at[h],
                recv_sem=vrecv.at[h],
                device_id=(my_x, right, my_z),
                device_id_type=pl.DeviceIdType.MESH,
            )
            k_recv_rdma.wait_recv()
            v_recv_rdma.wait_recv()
            k_send_rdma.wait_send()
            v_send_rdma.wait_send()

        for b in range(B):
            for hh in range(H):
                q = q_ref[b, :, hh, :].astype(jnp.bfloat16)
                s_parts = []
                for c in range(N_Y):
                    kc = kfull[c, b, :, hh, :]
                    s_parts.append(lax.dot_general(
                        q, kc, (((1,), (1,)), ((), ())),
                        preferred_element_type=jnp.float32,
                    ))
                s = jnp.concatenate(s_parts, axis=1) * SCALE
                m = jnp.max(s, axis=1, keepdims=True)
                p = jnp.exp(s - m)
                l = jnp.sum(p, axis=1, keepdims=True)
                pb = p.astype(jnp.bfloat16)
                o = None
                for c in range(N_Y):
                    vc = vfull[c, b, :, hh, :]
                    oc = lax.dot_general(
                        pb[:, c * S_PER:(c + 1) * S_PER], vc,
                        (((1,), (0,)), ((), ())),
                        preferred_element_type=jnp.float32,
                    )
                    o = oc if o is None else o + oc
                out_ref[b, :, hh, :] = o / l

    return pl.pallas_call(
        body,
        out_shape=jax.ShapeDtypeStruct((B, S_PER, H, D), jnp.float32),
        in_specs=[
            pl.BlockSpec(memory_space=pltpu.VMEM),
            pl.BlockSpec(memory_space=pltpu.VMEM),
            pl.BlockSpec(memory_space=pltpu.VMEM),
        ],
        out_specs=pl.BlockSpec(memory_space=pltpu.VMEM),
        scratch_shapes=[
            pltpu.VMEM((N_Y, B, S_PER, H, D), jnp.bfloat16),
            pltpu.VMEM((N_Y, B, S_PER, H, D), jnp.bfloat16),
            pltpu.SemaphoreType.DMA((N_Y - 1,)),
            pltpu.SemaphoreType.DMA((N_Y - 1,)),
            pltpu.SemaphoreType.DMA((N_Y - 1,)),
            pltpu.SemaphoreType.DMA((N_Y - 1,)),
        ],
        compiler_params=pltpu.CompilerParams(collective_id=0),
    )(Q, K, V)


# baseline (device time: 122327 ns/iter reference)
import jax
import jax.numpy as jnp
from jax import lax
from jax.experimental import pallas as pl
from jax.experimental.pallas import tpu as pltpu

N_Y = 4
B = 2
S_PER = 512
H = 8
D = 64
SCALE = D ** -0.5


def kernel(Q, K, V):
    Kb = K.astype(jnp.bfloat16).reshape(B, S_PER, H * D)
    Vb = V.astype(jnp.bfloat16).reshape(B, S_PER, H * D)

    def body(q_ref, k_ref, v_ref, out_ref,
             kfull, vfull, ksend, krecv, vsend, vrecv):
        my_x = lax.axis_index("x")
        my_y = lax.axis_index("y")
        my_z = lax.axis_index("z")
        right = (my_y + 1) % N_Y
        left = (my_y + N_Y - 1) % N_Y

        barrier = pltpu.get_barrier_semaphore()
        for nbr in (left, right):
            pl.semaphore_signal(
                barrier, inc=1,
                device_id=(my_x, nbr, my_z),
                device_id_type=pl.DeviceIdType.MESH,
            )
        pl.semaphore_wait(barrier, 2)

        kfull[pl.ds(my_y, 1)] = k_ref[...][None]
        vfull[pl.ds(my_y, 1)] = v_ref[...][None]

        for h in range(N_Y - 1):
            k_src = (my_y + N_Y - h) % N_Y
            k_rcv = (my_y + N_Y - h - 1) % N_Y
            v_src = (my_y + h) % N_Y
            v_rcv = (my_y + h + 1) % N_Y

            k_send_rdma = pltpu.make_async_remote_copy(
                src_ref=kfull.at[k_src],
                dst_ref=kfull.at[k_src],
                send_sem=ksend.at[h],
                recv_sem=krecv.at[h],
                device_id=(my_x, right, my_z),
                device_id_type=pl.DeviceIdType.MESH,
            )
            v_send_rdma = pltpu.make_async_remote_copy(
                src_ref=vfull.at[v_src],
                dst_ref=vfull.at[v_src],
                send_sem=vsend.at[h],
                recv_sem=vrecv.at[h],
                device_id=(my_x, left, my_z),
                device_id_type=pl.DeviceIdType.MESH,
            )
            k_send_rdma.start()
            v_send_rdma.start()

            k_recv_rdma = pltpu.make_async_remote_copy(
                src_ref=kfull.at[k_rcv],
                dst_ref=kfull.at[k_rcv],
                send_sem=ksend.at[h],
                recv_sem=krecv.at[h],
                device_id=(my_x, left, my_z),
                device_id_type=pl.DeviceIdType.MESH,
            )
            v_recv_rdma = pltpu.make_async_remote_copy(
                src_ref=vfull.at[v_rcv],
                dst_ref=vfull.at[v_rcv],
                send_sem=vsend.at[h],
                recv_sem=vrecv.at[h],
                device_id=(my_x, right, my_z),
                device_id_type=pl.DeviceIdType.MESH,
            )
            k_recv_rdma.wait_recv()
            v_recv_rdma.wait_recv()
            k_send_rdma.wait_send()
            v_send_rdma.wait_send()

        for b in range(B):
            for hh in range(H):
                q = q_ref[b, :, hh, :].astype(jnp.bfloat16)
                s_parts = []
                for c in range(N_Y):
                    kc = kfull[c, b, :, hh * D:(hh + 1) * D]
                    s_parts.append(lax.dot_general(
                        q, kc, (((1,), (1,)), ((), ())),
                        preferred_element_type=jnp.float32,
                    ) * SCALE)
                m = s_parts[0]
                for c in range(1, N_Y):
                    m = jnp.maximum(m, s_parts[c])
                m = jnp.max(m, axis=1, keepdims=True)
                o = None
                l = None
                for c in range(N_Y):
                    p = jnp.exp(s_parts[c] - m)
                    lc = jnp.sum(p, axis=1, keepdims=True)
                    vc = vfull[c, b, :, hh * D:(hh + 1) * D]
                    oc = lax.dot_general(
                        p.astype(jnp.bfloat16), vc,
                        (((1,), (0,)), ((), ())),
                        preferred_element_type=jnp.float32,
                    )
                    o = oc if o is None else o + oc
                    l = lc if l is None else l + lc
                out_ref[b, :, hh, :] = o / l

    return pl.pallas_call(
        body,
        out_shape=jax.ShapeDtypeStruct((B, S_PER, H, D), jnp.float32),
        in_specs=[
            pl.BlockSpec(memory_space=pltpu.VMEM),
            pl.BlockSpec(memory_space=pltpu.VMEM),
            pl.BlockSpec(memory_space=pltpu.VMEM),
        ],
        out_specs=pl.BlockSpec(memory_space=pltpu.VMEM),
        scratch_shapes=[
            pltpu.VMEM((N_Y, B, S_PER, H * D), jnp.bfloat16),
            pltpu.VMEM((N_Y, B, S_PER, H * D), jnp.bfloat16),
            pltpu.SemaphoreType.DMA((N_Y - 1,)),
            pltpu.SemaphoreType.DMA((N_Y - 1,)),
            pltpu.SemaphoreType.DMA((N_Y - 1,)),
            pltpu.SemaphoreType.DMA((N_Y - 1,)),
        ],
        compiler_params=pltpu.CompilerParams(
            collective_id=0,
            vmem_limit_bytes=60 * 1024 * 1024,
        ),
    )(Q, Kb, Vb)
